# baseline (speedup 1.0000x reference)
import functools
import math

import jax
import jax.numpy as jnp
from jax import lax
from jax.experimental import pallas as pl
from jax.experimental.pallas import tpu as pltpu
from jax.experimental.pallas import tpu_sc as plsc

CUTOFF = 5.0
H = 64
F = 64
NI = 3
G = 64

N_NODES = 50000
E_EDGES = 800000

N_TILES = 16
N_CORES = 2
EPT = 50176
CHUNKS = EPT // 128
E_PAD = EPT * N_TILES
N_ACC = 51200
APT = N_ACC // N_TILES

BN = 2000
BNN = 400
BEQ = 512


def _ssp(x):
    return jnp.maximum(x, 0.0) + jnp.log1p(jnp.exp(-jnp.abs(x))) - math.log(2.0)


def _emb_body(z_ref, emb_ref, w1_ref, h_ref, xjp_ref):
    z = z_ref[...]
    oh = (z == lax.broadcasted_iota(jnp.int32, (BN, 128), 1)).astype(jnp.float32)
    h = jnp.dot(oh, emb_ref[...], preferred_element_type=jnp.float32)
    h_ref[...] = h
    xp = jnp.dot(h, w1_ref[0], preferred_element_type=jnp.float32)
    xjp_ref[...] = jnp.concatenate(
        [xp, jnp.zeros((BN, 64), jnp.float32)], axis=1)


def _emb_call(z2, emb_p, cf_w1):
    n_blk = N_NODES // BN
    return pl.pallas_call(
        _emb_body,
        grid=(n_blk,),
        in_specs=[
            pl.BlockSpec((BN, 1), lambda n: (n, 0)),
            pl.BlockSpec((128, H), lambda n: (0, 0)),
            pl.BlockSpec((1, H, F), lambda n: (0, 0, 0)),
        ],
        out_specs=[
            pl.BlockSpec((BN, H), lambda n: (n, 0)),
            pl.BlockSpec((BN, 128), lambda n: (n, 0)),
        ],
        out_shape=[
            jax.ShapeDtypeStruct((N_NODES, H), jnp.float32),
            jax.ShapeDtypeStruct((N_NODES, 128), jnp.float32),
        ],
    )(z2, emb_p, cf_w1)


def _edge_body(e0_ref, e1_ref, e2_ref, e3_ref, w1_ref, b1_ref, w2_ref,
               b2_ref, w_ref):
    step = CUTOFF / (F - 1)
    off = lax.broadcasted_iota(jnp.int32, (1, F), 1).astype(jnp.float32) * step
    coeff = -0.5 / (step * step)
    lo = []
    hi = []
    for eref in (e0_ref, e1_ref, e2_ref, e3_ref):
        d = eref[0]
        ea = jnp.exp(coeff * (d - off) ** 2)
        t = _ssp(jnp.dot(ea, w1_ref[0], preferred_element_type=jnp.float32)
                 + b1_ref[0])
        t = jnp.dot(t, w2_ref[0], preferred_element_type=jnp.float32) + b2_ref[0]
        cwin = 0.5 * (jnp.cos(d * (math.pi / CUTOFF)) + 1.0)
        t = t * cwin
        lo.append(t[:, :32])
        hi.append(t[:, 32:])
    w_ref[0, 0] = jnp.concatenate(lo, axis=1)
    w_ref[0, 1] = jnp.concatenate(hi, axis=1)


def _edge_call(ew4t, mlp_w1, mlp_b1, mlp_w2, mlp_b2):
    e_blk = (E_PAD // 4) // BEQ
    qspec = lambda q: pl.BlockSpec((1, BEQ, 1), lambda i, e, q=q: (q, e, 0))
    return pl.pallas_call(
        _edge_body,
        grid=(NI, e_blk),
        in_specs=[
            qspec(0), qspec(1), qspec(2), qspec(3),
            pl.BlockSpec((1, F, F), lambda i, e: (i, 0, 0)),
            pl.BlockSpec((1, 1, F), lambda i, e: (i, 0, 0)),
            pl.BlockSpec((1, F, F), lambda i, e: (i, 0, 0)),
            pl.BlockSpec((1, 1, F), lambda i, e: (i, 0, 0)),
        ],
        out_specs=pl.BlockSpec((1, 2, BEQ, 128), lambda i, e: (i, 0, e, 0)),
        out_shape=jax.ShapeDtypeStruct((NI, 2, E_PAD // 4, 128), jnp.float32),
    )(ew4t, ew4t, ew4t, ew4t, mlp_w1, mlp_b1.reshape(NI, 1, F), mlp_w2,
      mlp_b2.reshape(NI, 1, F))


def _make_sc_agg(li):
    mesh = plsc.VectorSubcoreMesh(core_axis_name="c", subcore_axis_name="s")

    @functools.partial(
        pl.kernel,
        out_type=jax.ShapeDtypeStruct((N_CORES * N_ACC, 32), jnp.float32),
        mesh=mesh,
        scratch_types=[
            pltpu.VMEM((8, 128), jnp.int32),
            pltpu.VMEM((8, 128), jnp.int32),
            pltpu.VMEM((32, 128), jnp.float32),
            pltpu.VMEM((32, 128), jnp.float32),
            pltpu.VMEM((128, 32), jnp.float32),
            pltpu.VMEM_SHARED((N_ACC, 32), jnp.float32),
            pltpu.SemaphoreType.DMA,
        ],
    )
    def sc_agg(xjp_hbm, w_hbm, src_hbm, dst_hbm, zidx_hbm, agg_hbm,
               srcb, dstb, rows, wb, prod, acc, sem):
        c = lax.axis_index("c")
        s = lax.axis_index("s")
        zbase = s * APT

        @pl.loop(0, 128)
        def _z(k):
            prod[k, pl.ds(0, 16)] = jnp.zeros((16,), jnp.float32)
            prod[k, pl.ds(16, 16)] = jnp.zeros((16,), jnp.float32)

        @pl.loop(0, 4)
        def _zc8(zc8):
            pltpu.sync_copy(zidx_hbm.at[pl.ds(s * 32 + zc8 * 8, 8)], dstb)

            @pl.loop(0, 8)
            def _zc(zc):
                pltpu.sync_copy(prod, acc.at[dstb.at[zc]])

        plsc.subcore_barrier()

        w_row0 = (li * 2 + c) * (E_PAD // 4) + s * (EPT // 4)

        @pl.loop(0, CHUNKS // 8)
        def _super(it8):
            g8 = s * CHUNKS + it8 * 8
            pltpu.sync_copy(src_hbm.at[pl.ds(g8, 8)], srcb)
            pltpu.sync_copy(dst_hbm.at[pl.ds(g8, 8)], dstb)

            @pl.loop(0, 8)
            def _chunk(j):
                it = it8 * 8 + j
                pltpu.sync_copy(w_hbm.at[pl.ds(w_row0 + it * 32, 32)], wb)

                for hh in (0, 1, 2, 3):
                    pltpu.async_copy(
                        xjp_hbm.at[srcb.at[j, pl.ds(hh * 32, 32)]],
                        rows, sem).wait()
                    for cc in (0, 32):
                        @pl.when(c == cc // 32)
                        def _():
                            @pl.loop(0, 32)
                            def _mul(k):
                                kk = hh * 32 + k
                                r = hh * 8 + k // 4
                                cb = (k % 4) * 32
                                prod[kk, pl.ds(0, 16)] = (
                                    rows[k, pl.ds(cc, 16)]
                                    * wb[r, pl.ds(cb, 16)])
                                prod[kk, pl.ds(16, 16)] = (
                                    rows[k, pl.ds(cc + 16, 16)]
                                    * wb[r, pl.ds(cb + 16, 16)])

                @pl.loop(0, N_TILES)
                def _turn(t):
                    @pl.when(s == t)
                    def _():
                        pltpu.async_copy(prod, acc.at[dstb.at[j]], sem,
                                         add=True).wait()

                    plsc.subcore_barrier()

        plsc.subcore_barrier()

        obase = c * N_ACC + zbase

        @pl.loop(0, 4)
        def _f8(z8):
            pltpu.sync_copy(zidx_hbm.at[pl.ds(s * 32 + z8 * 8, 8)], dstb)
            nz = jnp.where(z8 == 3, 1, 8)

            @pl.loop(0, nz)
            def _f(z):
                pltpu.async_copy(acc.at[dstb.at[z]], prod, sem).wait()
                pltpu.sync_copy(
                    prod,
                    agg_hbm.at[pl.ds(obase + (z8 * 8 + z) * 128, 128)])

    return sc_agg


_SC_AGG_CACHE = {}


def _sc_agg_for(li):
    if li not in _SC_AGG_CACHE:
        _SC_AGG_CACHE[li] = _make_sc_agg(li)
    return _SC_AGG_CACHE[li]


def _node_body(h_ref, ga_ref, gb_ref, w2_ref, b2_ref, bw_ref, bb_ref,
               w1n_ref, h_out, xjp_ref):
    w2 = w2_ref[0]
    x = (jnp.dot(ga_ref[...], w2[:32, :], preferred_element_type=jnp.float32)
         + jnp.dot(gb_ref[...], w2[32:, :], preferred_element_type=jnp.float32)
         + b2_ref[0])
    x = _ssp(x)
    x = jnp.dot(x, bw_ref[0], preferred_element_type=jnp.float32) + bb_ref[0]
    hn = h_ref[...] + x
    h_out[...] = hn
    xp = jnp.dot(hn, w1n_ref[0], preferred_element_type=jnp.float32)
    xjp_ref[...] = jnp.concatenate(
        [xp, jnp.zeros((BNN, 64), jnp.float32)], axis=1)


def _node_call(li, h, agg2, cf_w2, cf_b2, blk_w, blk_b, cf_w1):
    n_blk = N_NODES // BNN
    return pl.pallas_call(
        _node_body,
        grid=(n_blk,),
        in_specs=[
            pl.BlockSpec((BNN, H), lambda n: (n, 0)),
            pl.BlockSpec((BNN, 32), lambda n: (n, 0)),
            pl.BlockSpec((BNN, 32), lambda n: (n + N_ACC // BNN, 0)),
            pl.BlockSpec((1, F, H), lambda n: (li, 0, 0)),
            pl.BlockSpec((1, 1, H), lambda n: (li, 0, 0)),
            pl.BlockSpec((1, H, H), lambda n: (li, 0, 0)),
            pl.BlockSpec((1, 1, H), lambda n: (li, 0, 0)),
            pl.BlockSpec((1, H, F), lambda n: (li + 1, 0, 0)),
        ],
        out_specs=[
            pl.BlockSpec((BNN, H), lambda n: (n, 0)),
            pl.BlockSpec((BNN, 128), lambda n: (n, 0)),
        ],
        out_shape=[
            jax.ShapeDtypeStruct((N_NODES, H), jnp.float32),
            jax.ShapeDtypeStruct((N_NODES, 128), jnp.float32),
        ],
    )(h, agg2, agg2, cf_w2, cf_b2.reshape(NI, 1, H), blk_w,
      blk_b.reshape(NI, 1, H), cf_w1)


def _final_body(h_ref, ga_ref, gb_ref, w2_ref, b2_ref, bw_ref, bb_ref,
                ow1_ref, ob1_ref, ow2_ref, ob2_ref, batch_ref, out_ref):
    w2 = w2_ref[0]
    x = (jnp.dot(ga_ref[...], w2[:32, :], preferred_element_type=jnp.float32)
         + jnp.dot(gb_ref[...], w2[32:, :], preferred_element_type=jnp.float32)
         + b2_ref[0])
    x = _ssp(x)
    x = jnp.dot(x, bw_ref[0], preferred_element_type=jnp.float32) + bb_ref[0]
    hn = h_ref[...] + x
    t = _ssp(jnp.dot(hn, ow1_ref[...], preferred_element_type=jnp.float32)
             + ob1_ref[...])
    y = jnp.sum(t * ow2_ref[...], axis=1, keepdims=True) + ob2_ref[...]
    oh = (batch_ref[...] == lax.broadcasted_iota(jnp.int32, (BNN, G), 1)
          ).astype(jnp.float32)
    contrib = jnp.sum(oh * y, axis=0, keepdims=True)

    @pl.when(pl.program_id(0) == 0)
    def _():
        out_ref[...] = jnp.zeros_like(out_ref)

    out_ref[...] += contrib


def _final_call(li, h, agg2, cf_w2, cf_b2, blk_w, blk_b,
                out_w1, out_b1, out_w2, out_b2, batch2):
    n_blk = N_NODES // BNN
    return pl.pallas_call(
        _final_body,
        grid=(n_blk,),
        in_specs=[
            pl.BlockSpec((BNN, H), lambda n: (n, 0)),
            pl.BlockSpec((BNN, 32), lambda n: (n, 0)),
            pl.BlockSpec((BNN, 32), lambda n: (n + N_ACC // BNN, 0)),
            pl.BlockSpec((1, F, H), lambda n: (li, 0, 0)),
            pl.BlockSpec((1, 1, H), lambda n: (li, 0, 0)),
            pl.BlockSpec((1, H, H), lambda n: (li, 0, 0)),
            pl.BlockSpec((1, 1, H), lambda n: (li, 0, 0)),
            pl.BlockSpec((H, H // 2), lambda n: (0, 0)),
            pl.BlockSpec((1, H // 2), lambda n: (0, 0)),
            pl.BlockSpec((1, H // 2), lambda n: (0, 0)),
            pl.BlockSpec((1, 1), lambda n: (0, 0)),
            pl.BlockSpec((BNN, 1), lambda n: (n, 0)),
        ],
        out_specs=pl.BlockSpec((1, G), lambda n: (0, 0)),
        out_shape=jax.ShapeDtypeStruct((1, G), jnp.float32),
    )(h, agg2, agg2, cf_w2, cf_b2.reshape(NI, 1, H), blk_w,
      blk_b.reshape(NI, 1, H), out_w1, out_b1.reshape(1, -1),
      out_w2.reshape(1, -1), out_b2.reshape(1, 1), batch2)


def kernel(z, edge_index, edge_attr, batch, emb, mlp_w1, mlp_b1, mlp_w2,
           mlp_b2, cf_w1, cf_w2, cf_b2, blk_w, blk_b, out_w1, out_b1,
           out_w2, out_b2):
    pad = E_PAD - E_EDGES
    src = edge_index[0].astype(jnp.int32)
    dst = edge_index[1].astype(jnp.int32)
    src_p = jnp.concatenate([src, jnp.zeros((pad,), jnp.int32)])
    dst_p = jnp.concatenate([dst, jnp.full((pad,), N_NODES, jnp.int32)])
    src2 = src_p.reshape(-1, 128)
    dst2 = dst_p.reshape(-1, 128)
    ew = edge_attr.reshape(-1)
    ew_p = jnp.concatenate([ew, jnp.zeros((pad,), jnp.float32)])
    ew4t = ew_p.reshape(-1, 4).T.reshape(4, E_PAD // 4, 1)
    emb_p = jnp.pad(emb, ((0, 128 - emb.shape[0]), (0, 0)))
    z2 = z.reshape(N_NODES, 1).astype(jnp.int32)
    batch2 = batch.reshape(N_NODES, 1).astype(jnp.int32)

    jrow = jnp.minimum(jnp.arange(32), 24)
    zidx = (jnp.arange(N_TILES)[:, None, None] * APT
            + jrow[None, :, None] * 128
            + jnp.arange(128)[None, None, :]).astype(jnp.int32)
    zidx = zidx.reshape(N_TILES * 32, 128)

    h, xjp = _emb_call(z2, emb_p, cf_w1)
    w_all = _edge_call(ew4t, mlp_w1, mlp_b1, mlp_w2, mlp_b2)
    w_flat = w_all.reshape(-1, 128)

    for i in range(NI):
        agg2 = _sc_agg_for(i)(xjp, w_flat, src2, dst2, zidx)
        if i < NI - 1:
            h, xjp = _node_call(i, h, agg2, cf_w2, cf_b2, blk_w, blk_b, cf_w1)
        else:
            out2d = _final_call(i, h, agg2, cf_w2, cf_b2, blk_w, blk_b,
                                out_w1, out_b1, out_w2, out_b2, batch2)
    return out2d.reshape(-1)


# speedup vs baseline: 1.3327x; 1.3327x over previous
import functools
import math

import jax
import jax.numpy as jnp
from jax import lax
from jax.experimental import pallas as pl
from jax.experimental.pallas import tpu as pltpu
from jax.experimental.pallas import tpu_sc as plsc

CUTOFF = 5.0
H = 64
F = 64
NI = 3
G = 64

N_NODES = 50000
E_EDGES = 800000

N_TILES = 16
N_CORES = 2
EPT = 50176
CHUNKS = EPT // 128
E_PAD = EPT * N_TILES
N_ACC = 51200
APT = N_ACC // N_TILES

BN = 2000
BNN = 400
BEQ = 512


def _ssp(x):
    return jnp.maximum(x, 0.0) + jnp.log1p(jnp.exp(-jnp.abs(x))) - math.log(2.0)


def _emb_body(z_ref, emb_ref, w1_ref, h_ref, xjp_ref):
    z = z_ref[...]
    oh = (z == lax.broadcasted_iota(jnp.int32, (BN, 128), 1)).astype(jnp.float32)
    h = jnp.dot(oh, emb_ref[...], preferred_element_type=jnp.float32)
    h_ref[...] = h
    xp = jnp.dot(h, w1_ref[0], preferred_element_type=jnp.float32)
    xjp_ref[...] = jnp.concatenate(
        [xp, jnp.zeros((BN, 64), jnp.float32)], axis=1)


def _emb_call(z2, emb_p, cf_w1):
    n_blk = N_NODES // BN
    return pl.pallas_call(
        _emb_body,
        grid=(n_blk,),
        in_specs=[
            pl.BlockSpec((BN, 1), lambda n: (n, 0)),
            pl.BlockSpec((128, H), lambda n: (0, 0)),
            pl.BlockSpec((1, H, F), lambda n: (0, 0, 0)),
        ],
        out_specs=[
            pl.BlockSpec((BN, H), lambda n: (n, 0)),
            pl.BlockSpec((BN, 128), lambda n: (n, 0)),
        ],
        out_shape=[
            jax.ShapeDtypeStruct((N_NODES, H), jnp.float32),
            jax.ShapeDtypeStruct((N_NODES, 128), jnp.float32),
        ],
    )(z2, emb_p, cf_w1)


def _edge_body(e0_ref, e1_ref, e2_ref, e3_ref, w1_ref, b1_ref, w2_ref,
               b2_ref, w_ref):
    step = CUTOFF / (F - 1)
    off = lax.broadcasted_iota(jnp.int32, (1, F), 1).astype(jnp.float32) * step
    coeff = -0.5 / (step * step)
    lo = []
    hi = []
    for eref in (e0_ref, e1_ref, e2_ref, e3_ref):
        d = eref[0]
        ea = jnp.exp(coeff * (d - off) ** 2)
        t = _ssp(jnp.dot(ea, w1_ref[0], preferred_element_type=jnp.float32)
                 + b1_ref[0])
        t = jnp.dot(t, w2_ref[0], preferred_element_type=jnp.float32) + b2_ref[0]
        cwin = 0.5 * (jnp.cos(d * (math.pi / CUTOFF)) + 1.0)
        t = t * cwin
        lo.append(t[:, :32])
        hi.append(t[:, 32:])
    w_ref[0, 0] = jnp.concatenate(lo, axis=1)
    w_ref[0, 1] = jnp.concatenate(hi, axis=1)


def _edge_call(ew4t, mlp_w1, mlp_b1, mlp_w2, mlp_b2):
    e_blk = (E_PAD // 4) // BEQ
    qspec = lambda q: pl.BlockSpec((1, BEQ, 1), lambda i, e, q=q: (q, e, 0))
    return pl.pallas_call(
        _edge_body,
        grid=(NI, e_blk),
        in_specs=[
            qspec(0), qspec(1), qspec(2), qspec(3),
            pl.BlockSpec((1, F, F), lambda i, e: (i, 0, 0)),
            pl.BlockSpec((1, 1, F), lambda i, e: (i, 0, 0)),
            pl.BlockSpec((1, F, F), lambda i, e: (i, 0, 0)),
            pl.BlockSpec((1, 1, F), lambda i, e: (i, 0, 0)),
        ],
        out_specs=pl.BlockSpec((1, 2, BEQ, 128), lambda i, e: (i, 0, e, 0)),
        out_shape=jax.ShapeDtypeStruct((NI, 2, E_PAD // 4, 128), jnp.float32),
    )(ew4t, ew4t, ew4t, ew4t, mlp_w1, mlp_b1.reshape(NI, 1, F), mlp_w2,
      mlp_b2.reshape(NI, 1, F))


def _make_sc_agg(li):
    mesh = plsc.VectorSubcoreMesh(core_axis_name="c", subcore_axis_name="s")

    @functools.partial(
        pl.kernel,
        out_type=jax.ShapeDtypeStruct((N_CORES * N_ACC, 32), jnp.float32),
        mesh=mesh,
        scratch_types=[
            pltpu.VMEM((8, 128), jnp.int32),
            pltpu.VMEM((8, 128), jnp.int32),
            pltpu.VMEM((32, 128), jnp.float32),
            pltpu.VMEM((32, 128), jnp.float32),
            pltpu.VMEM((128, 32), jnp.float32),
            pltpu.VMEM_SHARED((N_ACC, 32), jnp.float32),
            pltpu.SemaphoreType.DMA,
        ],
    )
    def sc_agg(xjp_hbm, w_hbm, src_hbm, dst_hbm, zidx_hbm, agg_hbm,
               srcb, dstb, rows, wb, prod, acc, sem):
        c = lax.axis_index("c")
        s = lax.axis_index("s")
        zbase = s * APT

        @pl.loop(0, 128)
        def _z(k):
            prod[k, pl.ds(0, 16)] = jnp.zeros((16,), jnp.float32)
            prod[k, pl.ds(16, 16)] = jnp.zeros((16,), jnp.float32)

        @pl.loop(0, 4)
        def _zc8(zc8):
            pltpu.sync_copy(zidx_hbm.at[pl.ds(s * 32 + zc8 * 8, 8)], dstb)

            @pl.loop(0, 8)
            def _zc(zc):
                pltpu.sync_copy(prod, acc.at[dstb.at[zc]])

        plsc.subcore_barrier()

        w_row0 = (li * 2 + c) * (E_PAD // 4) + s * (EPT // 4)

        @pl.loop(0, CHUNKS // 8)
        def _super(it8):
            g8 = s * CHUNKS + it8 * 8
            pltpu.sync_copy(src_hbm.at[pl.ds(g8, 8)], srcb)
            pltpu.sync_copy(dst_hbm.at[pl.ds(g8, 8)], dstb)

            @pl.loop(0, 8)
            def _chunk(j):
                it = it8 * 8 + j
                pltpu.sync_copy(w_hbm.at[pl.ds(w_row0 + it * 32, 32)], wb)

                for hh in (0, 1, 2, 3):
                    pltpu.async_copy(
                        xjp_hbm.at[srcb.at[j, pl.ds(hh * 32, 32)]],
                        rows, sem).wait()
                    for cc in (0, 32):
                        @pl.when(c == cc // 32)
                        def _():
                            @pl.loop(0, 32)
                            def _mul(k):
                                kk = hh * 32 + k
                                r = hh * 8 + k // 4
                                cb = (k % 4) * 32
                                prod[kk, pl.ds(0, 16)] = (
                                    rows[k, pl.ds(cc, 16)]
                                    * wb[r, pl.ds(cb, 16)])
                                prod[kk, pl.ds(16, 16)] = (
                                    rows[k, pl.ds(cc + 16, 16)]
                                    * wb[r, pl.ds(cb + 16, 16)])

                pltpu.async_copy(prod, acc.at[dstb.at[j]], sem,
                                 add=True).wait()

        plsc.subcore_barrier()

        obase = c * N_ACC + zbase

        @pl.loop(0, 4)
        def _f8(z8):
            pltpu.sync_copy(zidx_hbm.at[pl.ds(s * 32 + z8 * 8, 8)], dstb)
            nz = jnp.where(z8 == 3, 1, 8)

            @pl.loop(0, nz)
            def _f(z):
                pltpu.async_copy(acc.at[dstb.at[z]], prod, sem).wait()
                pltpu.sync_copy(
                    prod,
                    agg_hbm.at[pl.ds(obase + (z8 * 8 + z) * 128, 128)])

    return sc_agg


_SC_AGG_CACHE = {}


def _sc_agg_for(li):
    if li not in _SC_AGG_CACHE:
        _SC_AGG_CACHE[li] = _make_sc_agg(li)
    return _SC_AGG_CACHE[li]


def _node_body(h_ref, ga_ref, gb_ref, w2_ref, b2_ref, bw_ref, bb_ref,
               w1n_ref, h_out, xjp_ref):
    w2 = w2_ref[0]
    x = (jnp.dot(ga_ref[...], w2[:32, :], preferred_element_type=jnp.float32)
         + jnp.dot(gb_ref[...], w2[32:, :], preferred_element_type=jnp.float32)
         + b2_ref[0])
    x = _ssp(x)
    x = jnp.dot(x, bw_ref[0], preferred_element_type=jnp.float32) + bb_ref[0]
    hn = h_ref[...] + x
    h_out[...] = hn
    xp = jnp.dot(hn, w1n_ref[0], preferred_element_type=jnp.float32)
    xjp_ref[...] = jnp.concatenate(
        [xp, jnp.zeros((BNN, 64), jnp.float32)], axis=1)


def _node_call(li, h, agg2, cf_w2, cf_b2, blk_w, blk_b, cf_w1):
    n_blk = N_NODES // BNN
    return pl.pallas_call(
        _node_body,
        grid=(n_blk,),
        in_specs=[
            pl.BlockSpec((BNN, H), lambda n: (n, 0)),
            pl.BlockSpec((BNN, 32), lambda n: (n, 0)),
            pl.BlockSpec((BNN, 32), lambda n: (n + N_ACC // BNN, 0)),
            pl.BlockSpec((1, F, H), lambda n: (li, 0, 0)),
            pl.BlockSpec((1, 1, H), lambda n: (li, 0, 0)),
            pl.BlockSpec((1, H, H), lambda n: (li, 0, 0)),
            pl.BlockSpec((1, 1, H), lambda n: (li, 0, 0)),
            pl.BlockSpec((1, H, F), lambda n: (li + 1, 0, 0)),
        ],
        out_specs=[
            pl.BlockSpec((BNN, H), lambda n: (n, 0)),
            pl.BlockSpec((BNN, 128), lambda n: (n, 0)),
        ],
        out_shape=[
            jax.ShapeDtypeStruct((N_NODES, H), jnp.float32),
            jax.ShapeDtypeStruct((N_NODES, 128), jnp.float32),
        ],
    )(h, agg2, agg2, cf_w2, cf_b2.reshape(NI, 1, H), blk_w,
      blk_b.reshape(NI, 1, H), cf_w1)


def _final_body(h_ref, ga_ref, gb_ref, w2_ref, b2_ref, bw_ref, bb_ref,
                ow1_ref, ob1_ref, ow2_ref, ob2_ref, batch_ref, out_ref):
    w2 = w2_ref[0]
    x = (jnp.dot(ga_ref[...], w2[:32, :], preferred_element_type=jnp.float32)
         + jnp.dot(gb_ref[...], w2[32:, :], preferred_element_type=jnp.float32)
         + b2_ref[0])
    x = _ssp(x)
    x = jnp.dot(x, bw_ref[0], preferred_element_type=jnp.float32) + bb_ref[0]
    hn = h_ref[...] + x
    t = _ssp(jnp.dot(hn, ow1_ref[...], preferred_element_type=jnp.float32)
             + ob1_ref[...])
    y = jnp.sum(t * ow2_ref[...], axis=1, keepdims=True) + ob2_ref[...]
    oh = (batch_ref[...] == lax.broadcasted_iota(jnp.int32, (BNN, G), 1)
          ).astype(jnp.float32)
    contrib = jnp.sum(oh * y, axis=0, keepdims=True)

    @pl.when(pl.program_id(0) == 0)
    def _():
        out_ref[...] = jnp.zeros_like(out_ref)

    out_ref[...] += contrib


def _final_call(li, h, agg2, cf_w2, cf_b2, blk_w, blk_b,
                out_w1, out_b1, out_w2, out_b2, batch2):
    n_blk = N_NODES // BNN
    return pl.pallas_call(
        _final_body,
        grid=(n_blk,),
        in_specs=[
            pl.BlockSpec((BNN, H), lambda n: (n, 0)),
            pl.BlockSpec((BNN, 32), lambda n: (n, 0)),
            pl.BlockSpec((BNN, 32), lambda n: (n + N_ACC // BNN, 0)),
            pl.BlockSpec((1, F, H), lambda n: (li, 0, 0)),
            pl.BlockSpec((1, 1, H), lambda n: (li, 0, 0)),
            pl.BlockSpec((1, H, H), lambda n: (li, 0, 0)),
            pl.BlockSpec((1, 1, H), lambda n: (li, 0, 0)),
            pl.BlockSpec((H, H // 2), lambda n: (0, 0)),
            pl.BlockSpec((1, H // 2), lambda n: (0, 0)),
            pl.BlockSpec((1, H // 2), lambda n: (0, 0)),
            pl.BlockSpec((1, 1), lambda n: (0, 0)),
            pl.BlockSpec((BNN, 1), lambda n: (n, 0)),
        ],
        out_specs=pl.BlockSpec((1, G), lambda n: (0, 0)),
        out_shape=jax.ShapeDtypeStruct((1, G), jnp.float32),
    )(h, agg2, agg2, cf_w2, cf_b2.reshape(NI, 1, H), blk_w,
      blk_b.reshape(NI, 1, H), out_w1, out_b1.reshape(1, -1),
      out_w2.reshape(1, -1), out_b2.reshape(1, 1), batch2)


def kernel(z, edge_index, edge_attr, batch, emb, mlp_w1, mlp_b1, mlp_w2,
           mlp_b2, cf_w1, cf_w2, cf_b2, blk_w, blk_b, out_w1, out_b1,
           out_w2, out_b2):
    pad = E_PAD - E_EDGES
    src = edge_index[0].astype(jnp.int32)
    dst = edge_index[1].astype(jnp.int32)
    src_p = jnp.concatenate([src, jnp.zeros((pad,), jnp.int32)])
    dst_p = jnp.concatenate([dst, jnp.full((pad,), N_NODES, jnp.int32)])
    src2 = src_p.reshape(-1, 128)
    dst2 = dst_p.reshape(-1, 128)
    ew = edge_attr.reshape(-1)
    ew_p = jnp.concatenate([ew, jnp.zeros((pad,), jnp.float32)])
    ew4t = ew_p.reshape(-1, 4).T.reshape(4, E_PAD // 4, 1)
    emb_p = jnp.pad(emb, ((0, 128 - emb.shape[0]), (0, 0)))
    z2 = z.reshape(N_NODES, 1).astype(jnp.int32)
    batch2 = batch.reshape(N_NODES, 1).astype(jnp.int32)

    jrow = jnp.minimum(jnp.arange(32), 24)
    zidx = (jnp.arange(N_TILES)[:, None, None] * APT
            + jrow[None, :, None] * 128
            + jnp.arange(128)[None, None, :]).astype(jnp.int32)
    zidx = zidx.reshape(N_TILES * 32, 128)

    h, xjp = _emb_call(z2, emb_p, cf_w1)
    w_all = _edge_call(ew4t, mlp_w1, mlp_b1, mlp_w2, mlp_b2)
    w_flat = w_all.reshape(-1, 128)

    for i in range(NI):
        agg2 = _sc_agg_for(i)(xjp, w_flat, src2, dst2, zidx)
        if i < NI - 1:
            h, xjp = _node_call(i, h, agg2, cf_w2, cf_b2, blk_w, blk_b, cf_w1)
        else:
            out2d = _final_call(i, h, agg2, cf_w2, cf_b2, blk_w, blk_b,
                                out_w1, out_b1, out_w2, out_b2, batch2)
    return out2d.reshape(-1)
